# Initial kernel scaffold; baseline (speedup 1.0000x reference)
import functools

import jax
import jax.numpy as jnp
from jax import lax
from jax.experimental import pallas as pl
from jax.experimental.pallas import tpu as pltpu
from jax.experimental.pallas import tpu_sc as plsc

N = 10000
D = 128
E = 320000
TEMP = 0.2

NC = 2
NS = 16
NW = NC * NS

CHUNK = 128
CHUNKS_PER_TILE = 79
E_PER_TILE = CHUNK * CHUNKS_PER_TILE
E_PAD = NW * E_PER_TILE
SINK = N
ACC_ROWS = 10016

ROW_BLK = 1000

_mesh = plsc.VectorSubcoreMesh(core_axis_name="c", subcore_axis_name="s")


def _deg_body(dstp, zeros1, out, idx_d, ones_v, acc):
    c = lax.axis_index("c")
    s = lax.axis_index("s")
    w = c * NS + s

    @pl.when(s < 4)
    def _():
        pltpu.sync_copy(zeros1.at[pl.ds(s * 2504, 2504)],
                        acc.at[pl.ds(s * 2504, 2504)])

    for i in range(CHUNK // 16):
        ones_v[pl.ds(i * 16, 16)] = jnp.ones((16,), jnp.float32)
    plsc.subcore_barrier()

    def body(k, carry):
        off = w * E_PER_TILE + k * CHUNK
        pltpu.sync_copy(dstp.at[pl.ds(off, CHUNK)], idx_d)
        pltpu.sync_copy(ones_v, acc.at[idx_d], add=True)
        return carry

    lax.fori_loop(0, CHUNKS_PER_TILE, body, 0)
    plsc.subcore_barrier()

    @pl.when(s < 4)
    def _():
        pltpu.sync_copy(acc.at[pl.ds(s * 2504, 2504)],
                        out.at[c, pl.ds(s * 2504, 2504)])


_deg_call = pl.kernel(
    _deg_body,
    out_type=jax.ShapeDtypeStruct((NC, ACC_ROWS), jnp.float32),
    mesh=_mesh,
    scratch_types=[
        pltpu.VMEM((CHUNK,), jnp.int32),
        pltpu.VMEM((CHUNK,), jnp.float32),
        pltpu.VMEM_SHARED((ACC_ROWS,), jnp.float32),
    ],
)


def _agg_body(g, srcp, dstp, zeros2, out, idx_s, idx_d, rows, sem, acc):
    c = lax.axis_index("c")
    s = lax.axis_index("s")
    w = c * NS + s

    pltpu.sync_copy(zeros2.at[pl.ds(s * 626, 626)],
                    acc.at[pl.ds(s * 626, 626)])
    plsc.subcore_barrier()

    def body(k, carry):
        off = w * E_PER_TILE + k * CHUNK
        pltpu.sync_copy(srcp.at[pl.ds(off, CHUNK)], idx_s)
        pltpu.async_copy(g.at[idx_s], rows, sem).wait()
        pltpu.sync_copy(dstp.at[pl.ds(off, CHUNK)], idx_d)
        pltpu.sync_copy(rows, acc.at[idx_d], add=True)
        return carry

    lax.fori_loop(0, CHUNKS_PER_TILE, body, 0)
    plsc.subcore_barrier()
    pltpu.sync_copy(acc.at[pl.ds(s * 625, 625)],
                    out.at[c, pl.ds(s * 625, 625)])


_agg_call = pl.kernel(
    _agg_body,
    out_type=jax.ShapeDtypeStruct((NC, N, D), jnp.float32),
    mesh=_mesh,
    scratch_types=[
        pltpu.VMEM((CHUNK,), jnp.int32),
        pltpu.VMEM((CHUNK,), jnp.int32),
        pltpu.VMEM((CHUNK, D), jnp.float32),
        pltpu.SemaphoreType.DMA,
        pltpu.VMEM_SHARED((ACC_ROWS, D), jnp.float32),
    ],
)


def _mm_body(x_ref, w_ref, o_ref):
    o_ref[...] = jnp.dot(x_ref[...], w_ref[...],
                         precision=lax.Precision.HIGHEST,
                         preferred_element_type=jnp.float32)


_mm_call = pl.pallas_call(
    _mm_body,
    grid=(N // ROW_BLK,),
    in_specs=[
        pl.BlockSpec((ROW_BLK, D), lambda i: (i, 0)),
        pl.BlockSpec((D, D), lambda i: (0, 0)),
    ],
    out_specs=pl.BlockSpec((ROW_BLK, D), lambda i: (i, 0)),
    out_shape=jax.ShapeDtypeStruct((N, D), jnp.float32),
)


def _scale_body(d0_ref, d1_ref, h_ref, dinv_ref, g_ref):
    deg = d0_ref[...] + d1_ref[...] + 1.0
    dinv = lax.rsqrt(deg)
    dinv_ref[...] = dinv
    g_ref[...] = h_ref[...] * dinv


_scale_call = pl.pallas_call(
    _scale_body,
    grid=(N // ROW_BLK,),
    in_specs=[
        pl.BlockSpec((ROW_BLK, 1), lambda i: (i, 0)),
        pl.BlockSpec((ROW_BLK, 1), lambda i: (i, 0)),
        pl.BlockSpec((ROW_BLK, D), lambda i: (i, 0)),
    ],
    out_specs=[
        pl.BlockSpec((ROW_BLK, 1), lambda i: (i, 0)),
        pl.BlockSpec((ROW_BLK, D), lambda i: (i, 0)),
    ],
    out_shape=[
        jax.ShapeDtypeStruct((N, 1), jnp.float32),
        jax.ShapeDtypeStruct((N, D), jnp.float32),
    ],
)


def _layer2_body(a_ref, g1_ref, dinv_ref, b1_ref, w2_ref, g2_ref):
    acc = a_ref[0] + a_ref[1]
    z = jnp.maximum(dinv_ref[...] * (acc + g1_ref[...]) + b1_ref[...], 0.0)
    h2 = jnp.dot(z, w2_ref[...],
                 precision=lax.Precision.HIGHEST,
                 preferred_element_type=jnp.float32)
    g2_ref[...] = h2 * dinv_ref[...]


_layer2_call = pl.pallas_call(
    _layer2_body,
    grid=(N // ROW_BLK,),
    in_specs=[
        pl.BlockSpec((NC, ROW_BLK, D), lambda i: (0, i, 0)),
        pl.BlockSpec((ROW_BLK, D), lambda i: (i, 0)),
        pl.BlockSpec((ROW_BLK, 1), lambda i: (i, 0)),
        pl.BlockSpec((1, D), lambda i: (0, 0)),
        pl.BlockSpec((D, D), lambda i: (0, 0)),
    ],
    out_specs=pl.BlockSpec((ROW_BLK, D), lambda i: (i, 0)),
    out_shape=jax.ShapeDtypeStruct((N, D), jnp.float32),
)


def _out_body(a_ref, g2_ref, dinv_ref, b2_ref, o_ref):
    acc = a_ref[0] + a_ref[1]
    y = (dinv_ref[...] * (acc + g2_ref[...]) + b2_ref[...]) / TEMP
    m = jnp.max(y, axis=1, keepdims=True)
    lse = jnp.log(jnp.sum(jnp.exp(y - m), axis=1, keepdims=True)) + m
    o_ref[...] = y - lse


_out_call = pl.pallas_call(
    _out_body,
    grid=(N // ROW_BLK,),
    in_specs=[
        pl.BlockSpec((NC, ROW_BLK, D), lambda i: (0, i, 0)),
        pl.BlockSpec((ROW_BLK, D), lambda i: (i, 0)),
        pl.BlockSpec((ROW_BLK, 1), lambda i: (i, 0)),
        pl.BlockSpec((1, D), lambda i: (0, 0)),
    ],
    out_specs=pl.BlockSpec((ROW_BLK, D), lambda i: (i, 0)),
    out_shape=jax.ShapeDtypeStruct((N, D), jnp.float32),
)


def kernel(x, edge_index, W1, b1, W2, b2):
    src = edge_index[0].astype(jnp.int32)
    dst = edge_index[1].astype(jnp.int32)
    srcp = jnp.concatenate([src, jnp.zeros((E_PAD - E,), jnp.int32)])
    dstp = jnp.concatenate([dst, jnp.full((E_PAD - E,), SINK, jnp.int32)])
    zeros1 = jnp.zeros((ACC_ROWS,), jnp.float32)
    zeros2 = jnp.zeros((ACC_ROWS, D), jnp.float32)

    degs = _deg_call(dstp, zeros1)
    h1 = _mm_call(x, W1)
    dinv, g1 = _scale_call(degs[0].reshape(ACC_ROWS, 1),
                           degs[1].reshape(ACC_ROWS, 1), h1)
    acc1 = _agg_call(g1, srcp, dstp, zeros2)
    g2 = _layer2_call(acc1, g1, dinv, b1.reshape(1, D), W2)
    acc2 = _agg_call(g2, srcp, dstp, zeros2)
    return _out_call(acc2, g2, dinv, b2.reshape(1, D))


# speedup vs baseline: 10.8403x; 10.8403x over previous
import functools

import jax
import jax.numpy as jnp
from jax import lax
from jax.experimental import pallas as pl
from jax.experimental.pallas import tpu as pltpu
from jax.experimental.pallas import tpu_sc as plsc

N = 10000
D = 128
E = 320000
TEMP = 0.2

NC = 2
NS = 16
NW = NC * NS

CHUNK = 128
CHUNKS_PER_TILE = 79
E_PER_TILE = CHUNK * CHUNKS_PER_TILE
E_PAD = NW * E_PER_TILE
SINK = N
ACC_ROWS = 10240
ACC_SLICE = ACC_ROWS // NS

ROW_BLK = 1000

def _deg_body(dstp, zeros1, out0, out1, idx_d, ones_v, acc):
    c = lax.axis_index("c")
    s = lax.axis_index("s")
    w = c * NS + s

    pltpu.sync_copy(zeros1.at[pl.ds(s * ACC_SLICE, ACC_SLICE)],
                    acc.at[pl.ds(s * ACC_SLICE, ACC_SLICE)])

    for i in range(CHUNK // 16):
        ones_v[pl.ds(i * 16, 16)] = jnp.ones((16,), jnp.float32)
    plsc.subcore_barrier()

    def body(k, carry):
        off = w * E_PER_TILE + k * CHUNK
        pltpu.sync_copy(dstp.at[pl.ds(off, CHUNK)], idx_d)
        pltpu.sync_copy(ones_v, acc.at[idx_d], add=True)
        return carry

    lax.fori_loop(0, CHUNKS_PER_TILE, body, 0)
    plsc.subcore_barrier()

    @pl.when(c == 0)
    def _():
        pltpu.sync_copy(acc.at[pl.ds(s * ACC_SLICE, ACC_SLICE)],
                        out0.at[pl.ds(s * ACC_SLICE, ACC_SLICE)])

    @pl.when(c == 1)
    def _():
        pltpu.sync_copy(acc.at[pl.ds(s * ACC_SLICE, ACC_SLICE)],
                        out1.at[pl.ds(s * ACC_SLICE, ACC_SLICE)])


@functools.cache
def _deg_call():
    mesh = plsc.VectorSubcoreMesh(core_axis_name="c", subcore_axis_name="s")
    return pl.kernel(
        _deg_body,
        out_type=[jax.ShapeDtypeStruct((ACC_ROWS,), jnp.float32),
                  jax.ShapeDtypeStruct((ACC_ROWS,), jnp.float32)],
        mesh=mesh,
        scratch_types=[
            pltpu.VMEM((CHUNK,), jnp.int32),
            pltpu.VMEM((CHUNK,), jnp.float32),
            pltpu.VMEM_SHARED((ACC_ROWS,), jnp.float32),
        ],
    )


def _agg_body(g, srcp, dstp, zeros2, out0, out1, idx_s, idx_d, rows, sem, acc):
    c = lax.axis_index("c")
    s = lax.axis_index("s")
    w = c * NS + s

    pltpu.sync_copy(zeros2.at[pl.ds(s * ACC_SLICE, ACC_SLICE)],
                    acc.at[pl.ds(s * ACC_SLICE, ACC_SLICE)])
    plsc.subcore_barrier()

    def body(k, carry):
        off = w * E_PER_TILE + k * CHUNK
        pltpu.sync_copy(srcp.at[pl.ds(off, CHUNK)], idx_s)
        pltpu.async_copy(g.at[idx_s], rows, sem).wait()
        pltpu.sync_copy(dstp.at[pl.ds(off, CHUNK)], idx_d)
        pltpu.sync_copy(rows, acc.at[idx_d], add=True)
        return carry

    lax.fori_loop(0, CHUNKS_PER_TILE, body, 0)
    plsc.subcore_barrier()

    @pl.when(jnp.logical_and(c == 0, s < 10))
    def _():
        pltpu.sync_copy(acc.at[pl.ds(s * 1000, 1000)],
                        out0.at[pl.ds(s * 1000, 1000)])

    @pl.when(jnp.logical_and(c == 1, s < 10))
    def _():
        pltpu.sync_copy(acc.at[pl.ds(s * 1000, 1000)],
                        out1.at[pl.ds(s * 1000, 1000)])


@functools.cache
def _agg_call():
    mesh = plsc.VectorSubcoreMesh(core_axis_name="c", subcore_axis_name="s")
    return pl.kernel(
        _agg_body,
        out_type=[jax.ShapeDtypeStruct((N, D), jnp.float32),
                  jax.ShapeDtypeStruct((N, D), jnp.float32)],
        mesh=mesh,
        scratch_types=[
            pltpu.VMEM((CHUNK,), jnp.int32),
            pltpu.VMEM((CHUNK,), jnp.int32),
            pltpu.VMEM((CHUNK, D), jnp.float32),
            pltpu.SemaphoreType.DMA,
            pltpu.VMEM_SHARED((ACC_ROWS, D), jnp.float32),
        ],
    )


def _mm_body(x_ref, w_ref, o_ref):
    o_ref[...] = jnp.dot(x_ref[...], w_ref[...],
                         precision=lax.Precision.HIGHEST,
                         preferred_element_type=jnp.float32)


_mm_call = pl.pallas_call(
    _mm_body,
    grid=(N // ROW_BLK,),
    in_specs=[
        pl.BlockSpec((ROW_BLK, D), lambda i: (i, 0)),
        pl.BlockSpec((D, D), lambda i: (0, 0)),
    ],
    out_specs=pl.BlockSpec((ROW_BLK, D), lambda i: (i, 0)),
    out_shape=jax.ShapeDtypeStruct((N, D), jnp.float32),
)


def _scale_body(d0_ref, d1_ref, h_ref, dinv_ref, g_ref):
    deg = d0_ref[...] + d1_ref[...] + 1.0
    dinv = lax.rsqrt(deg)
    dinv_ref[...] = dinv
    g_ref[...] = h_ref[...] * dinv


_scale_call = pl.pallas_call(
    _scale_body,
    grid=(N // ROW_BLK,),
    in_specs=[
        pl.BlockSpec((ROW_BLK, 1), lambda i: (i, 0)),
        pl.BlockSpec((ROW_BLK, 1), lambda i: (i, 0)),
        pl.BlockSpec((ROW_BLK, D), lambda i: (i, 0)),
    ],
    out_specs=[
        pl.BlockSpec((ROW_BLK, 1), lambda i: (i, 0)),
        pl.BlockSpec((ROW_BLK, D), lambda i: (i, 0)),
    ],
    out_shape=[
        jax.ShapeDtypeStruct((N, 1), jnp.float32),
        jax.ShapeDtypeStruct((N, D), jnp.float32),
    ],
)


def _layer2_body(a0_ref, a1_ref, g1_ref, dinv_ref, b1_ref, w2_ref, g2_ref):
    acc = a0_ref[...] + a1_ref[...]
    z = jnp.maximum(dinv_ref[...] * (acc + g1_ref[...]) + b1_ref[...], 0.0)
    h2 = jnp.dot(z, w2_ref[...],
                 precision=lax.Precision.HIGHEST,
                 preferred_element_type=jnp.float32)
    g2_ref[...] = h2 * dinv_ref[...]


_layer2_call = pl.pallas_call(
    _layer2_body,
    grid=(N // ROW_BLK,),
    in_specs=[
        pl.BlockSpec((ROW_BLK, D), lambda i: (i, 0)),
        pl.BlockSpec((ROW_BLK, D), lambda i: (i, 0)),
        pl.BlockSpec((ROW_BLK, D), lambda i: (i, 0)),
        pl.BlockSpec((ROW_BLK, 1), lambda i: (i, 0)),
        pl.BlockSpec((1, D), lambda i: (0, 0)),
        pl.BlockSpec((D, D), lambda i: (0, 0)),
    ],
    out_specs=pl.BlockSpec((ROW_BLK, D), lambda i: (i, 0)),
    out_shape=jax.ShapeDtypeStruct((N, D), jnp.float32),
)


def _out_body(a0_ref, a1_ref, g2_ref, dinv_ref, b2_ref, o_ref):
    acc = a0_ref[...] + a1_ref[...]
    y = (dinv_ref[...] * (acc + g2_ref[...]) + b2_ref[...]) / TEMP
    m = jnp.max(y, axis=1, keepdims=True)
    lse = jnp.log(jnp.sum(jnp.exp(y - m), axis=1, keepdims=True)) + m
    o_ref[...] = y - lse


_out_call = pl.pallas_call(
    _out_body,
    grid=(N // ROW_BLK,),
    in_specs=[
        pl.BlockSpec((ROW_BLK, D), lambda i: (i, 0)),
        pl.BlockSpec((ROW_BLK, D), lambda i: (i, 0)),
        pl.BlockSpec((ROW_BLK, D), lambda i: (i, 0)),
        pl.BlockSpec((ROW_BLK, 1), lambda i: (i, 0)),
        pl.BlockSpec((1, D), lambda i: (0, 0)),
    ],
    out_specs=pl.BlockSpec((ROW_BLK, D), lambda i: (i, 0)),
    out_shape=jax.ShapeDtypeStruct((N, D), jnp.float32),
)


def kernel(x, edge_index, W1, b1, W2, b2):
    src = edge_index[0].astype(jnp.int32)
    dst = edge_index[1].astype(jnp.int32)
    srcp = jnp.concatenate([src, jnp.zeros((E_PAD - E,), jnp.int32)])
    dstp = jnp.concatenate([dst, jnp.full((E_PAD - E,), SINK, jnp.int32)])
    zeros1 = jnp.zeros((ACC_ROWS,), jnp.float32)
    zeros2 = jnp.zeros((ACC_ROWS, D), jnp.float32)

    deg0, deg1 = _deg_call()(dstp, zeros1)
    h1 = _mm_call(x, W1)
    dinv, g1 = _scale_call(deg0.reshape(ACC_ROWS, 1),
                           deg1.reshape(ACC_ROWS, 1), h1)
    a10, a11 = _agg_call()(g1, srcp, dstp, zeros2)
    g2 = _layer2_call(a10, a11, g1, dinv, b1.reshape(1, D), W2)
    a20, a21 = _agg_call()(g2, srcp, dstp, zeros2)
    return _out_call(a20, a21, g2, dinv, b2.reshape(1, D))


# speedup vs baseline: 15.1373x; 1.3964x over previous
import functools

import jax
import jax.numpy as jnp
from jax import lax
from jax.experimental import pallas as pl
from jax.experimental.pallas import tpu as pltpu
from jax.experimental.pallas import tpu_sc as plsc

N = 10000
D = 128
E = 320000
TEMP = 0.2

NC = 2
NS = 16
NW = NC * NS

CHUNK = 128
CPT = 79
E_PAD = NW * CPT * CHUNK
SINK = N
ACC_ROWS = 10048
DEG_PAD = 10240
PAIRS = 39

ROW_BLK = 1000


def _deg_body(dstp3, zeros1, out0, out1, idx_d_all, ones_v, sem, acc):
    c = lax.axis_index("c")
    s = lax.axis_index("s")
    w = c * NS + s

    pltpu.sync_copy(zeros1.at[pl.ds(s * 640, 640)],
                    acc.at[pl.ds(s * 640, 640)])
    pltpu.sync_copy(dstp3.at[w], idx_d_all)
    for i in range(CHUNK // 16):
        ones_v[pl.ds(i * 16, 16)] = jnp.ones((16,), jnp.float32)
    plsc.subcore_barrier()

    def body(k, carry):
        pltpu.async_copy(ones_v, acc.at[idx_d_all.at[k]], sem, add=True)
        return carry

    lax.fori_loop(0, CPT, body, 0)

    def drain(k, carry):
        pltpu.make_async_copy(ones_v, acc.at[idx_d_all.at[0]], sem).wait()
        return carry

    lax.fori_loop(0, CPT, drain, 0)
    plsc.subcore_barrier()

    @pl.when(c == 0)
    def _():
        pltpu.sync_copy(acc.at[pl.ds(s * 640, 640)],
                        out0.at[pl.ds(s * 640, 640)])

    @pl.when(c == 1)
    def _():
        pltpu.sync_copy(acc.at[pl.ds(s * 640, 640)],
                        out1.at[pl.ds(s * 640, 640)])


@functools.cache
def _deg_call():
    mesh = plsc.VectorSubcoreMesh(core_axis_name="c", subcore_axis_name="s")
    return pl.kernel(
        _deg_body,
        out_type=[jax.ShapeDtypeStruct((DEG_PAD,), jnp.float32),
                  jax.ShapeDtypeStruct((DEG_PAD,), jnp.float32)],
        mesh=mesh,
        scratch_types=[
            pltpu.VMEM((CPT, CHUNK), jnp.int32),
            pltpu.VMEM((CHUNK,), jnp.float32),
            pltpu.SemaphoreType.DMA,
            pltpu.VMEM_SHARED((DEG_PAD,), jnp.float32),
        ],
    )


def _agg_body(g, srcp3, dstp3, zeros2, out0, out1,
              idx_d_all, is0, is1, r0, r1,
              si0, si1, sg0, sg1, ss0, ss1, acc):
    c = lax.axis_index("c")
    s = lax.axis_index("s")
    w = c * NS + s

    @pl.when(s < 10)
    def _():
        pltpu.sync_copy(zeros2, acc.at[pl.ds(s * 1000, 1000)])

    pltpu.sync_copy(dstp3.at[w], idx_d_all)
    plsc.subcore_barrier()

    def wait_gather(isx, rx, sgx):
        pltpu.make_async_copy(g.at[isx], rx, sgx).wait()

    def start_scatter(k, rx, ssx):
        pltpu.async_copy(rx, acc.at[idx_d_all.at[k]], ssx, add=True)

    def wait_scatter(rx, ssx):
        pltpu.make_async_copy(rx, acc.at[idx_d_all.at[0]], ssx).wait()

    def wait_idx(isx, six):
        pltpu.make_async_copy(srcp3.at[w, 0], isx, six).wait()

    pltpu.sync_copy(srcp3.at[w, 0], is0)
    pltpu.async_copy(g.at[is0], r0, sg0)
    pltpu.sync_copy(srcp3.at[w, 1], is1)
    wait_gather(is0, r0, sg0)
    start_scatter(0, r0, ss0)
    pltpu.async_copy(srcp3.at[w, 2], is0, si0)
    pltpu.async_copy(g.at[is1], r1, sg1)

    def pair(kk, carry):
        k0 = 2 * kk + 1
        k1 = 2 * kk + 2
        wait_gather(is1, r1, sg1)
        start_scatter(k0, r1, ss1)
        wait_scatter(r0, ss0)
        wait_idx(is0, si0)
        pltpu.async_copy(g.at[is0], r0, sg0)

        @pl.when(kk < PAIRS - 1)
        def _():
            pltpu.async_copy(srcp3.at[w, k0 + 2], is1, si1)

        wait_gather(is0, r0, sg0)
        start_scatter(k1, r0, ss0)
        wait_scatter(r1, ss1)

        @pl.when(kk < PAIRS - 1)
        def _():
            wait_idx(is1, si1)
            pltpu.async_copy(g.at[is1], r1, sg1)
            pltpu.async_copy(srcp3.at[w, k1 + 2], is0, si0)

        return carry

    lax.fori_loop(0, PAIRS, pair, 0)
    wait_scatter(r0, ss0)
    plsc.subcore_barrier()

    @pl.when(jnp.logical_and(c == 0, s < 10))
    def _():
        pltpu.sync_copy(acc.at[pl.ds(s * 1000, 1000)],
                        out0.at[pl.ds(s * 1000, 1000)])

    @pl.when(jnp.logical_and(c == 1, s < 10))
    def _():
        pltpu.sync_copy(acc.at[pl.ds(s * 1000, 1000)],
                        out1.at[pl.ds(s * 1000, 1000)])


@functools.cache
def _agg_call():
    mesh = plsc.VectorSubcoreMesh(core_axis_name="c", subcore_axis_name="s")
    return pl.kernel(
        _agg_body,
        out_type=[jax.ShapeDtypeStruct((N, D), jnp.float32),
                  jax.ShapeDtypeStruct((N, D), jnp.float32)],
        mesh=mesh,
        scratch_types=[
            pltpu.VMEM((CPT, CHUNK), jnp.int32),
            pltpu.VMEM((CHUNK,), jnp.int32),
            pltpu.VMEM((CHUNK,), jnp.int32),
            pltpu.VMEM((CHUNK, D), jnp.float32),
            pltpu.VMEM((CHUNK, D), jnp.float32),
            pltpu.SemaphoreType.DMA,
            pltpu.SemaphoreType.DMA,
            pltpu.SemaphoreType.DMA,
            pltpu.SemaphoreType.DMA,
            pltpu.SemaphoreType.DMA,
            pltpu.SemaphoreType.DMA,
            pltpu.VMEM_SHARED((ACC_ROWS, D), jnp.float32),
        ],
    )


def _mm_body(x_ref, w_ref, o_ref):
    o_ref[...] = jnp.dot(x_ref[...], w_ref[...],
                         precision=lax.Precision.HIGHEST,
                         preferred_element_type=jnp.float32)


_mm_call = pl.pallas_call(
    _mm_body,
    grid=(N // ROW_BLK,),
    in_specs=[
        pl.BlockSpec((ROW_BLK, D), lambda i: (i, 0)),
        pl.BlockSpec((D, D), lambda i: (0, 0)),
    ],
    out_specs=pl.BlockSpec((ROW_BLK, D), lambda i: (i, 0)),
    out_shape=jax.ShapeDtypeStruct((N, D), jnp.float32),
)


def _scale_body(d0_ref, d1_ref, h_ref, dinv_ref, g_ref):
    deg = d0_ref[...] + d1_ref[...] + 1.0
    dinv = lax.rsqrt(deg)
    dinv_ref[...] = dinv
    g_ref[...] = h_ref[...] * dinv


_scale_call = pl.pallas_call(
    _scale_body,
    grid=(N // ROW_BLK,),
    in_specs=[
        pl.BlockSpec((ROW_BLK, 1), lambda i: (i, 0)),
        pl.BlockSpec((ROW_BLK, 1), lambda i: (i, 0)),
        pl.BlockSpec((ROW_BLK, D), lambda i: (i, 0)),
    ],
    out_specs=[
        pl.BlockSpec((ROW_BLK, 1), lambda i: (i, 0)),
        pl.BlockSpec((ROW_BLK, D), lambda i: (i, 0)),
    ],
    out_shape=[
        jax.ShapeDtypeStruct((N, 1), jnp.float32),
        jax.ShapeDtypeStruct((N, D), jnp.float32),
    ],
)


def _layer2_body(a0_ref, a1_ref, g1_ref, dinv_ref, b1_ref, w2_ref, g2_ref):
    acc = a0_ref[...] + a1_ref[...]
    z = jnp.maximum(dinv_ref[...] * (acc + g1_ref[...]) + b1_ref[...], 0.0)
    h2 = jnp.dot(z, w2_ref[...],
                 precision=lax.Precision.HIGHEST,
                 preferred_element_type=jnp.float32)
    g2_ref[...] = h2 * dinv_ref[...]


_layer2_call = pl.pallas_call(
    _layer2_body,
    grid=(N // ROW_BLK,),
    in_specs=[
        pl.BlockSpec((ROW_BLK, D), lambda i: (i, 0)),
        pl.BlockSpec((ROW_BLK, D), lambda i: (i, 0)),
        pl.BlockSpec((ROW_BLK, D), lambda i: (i, 0)),
        pl.BlockSpec((ROW_BLK, 1), lambda i: (i, 0)),
        pl.BlockSpec((1, D), lambda i: (0, 0)),
        pl.BlockSpec((D, D), lambda i: (0, 0)),
    ],
    out_specs=pl.BlockSpec((ROW_BLK, D), lambda i: (i, 0)),
    out_shape=jax.ShapeDtypeStruct((N, D), jnp.float32),
)


def _out_body(a0_ref, a1_ref, g2_ref, dinv_ref, b2_ref, o_ref):
    acc = a0_ref[...] + a1_ref[...]
    y = (dinv_ref[...] * (acc + g2_ref[...]) + b2_ref[...]) / TEMP
    m = jnp.max(y, axis=1, keepdims=True)
    lse = jnp.log(jnp.sum(jnp.exp(y - m), axis=1, keepdims=True)) + m
    o_ref[...] = y - lse


_out_call = pl.pallas_call(
    _out_body,
    grid=(N // ROW_BLK,),
    in_specs=[
        pl.BlockSpec((ROW_BLK, D), lambda i: (i, 0)),
        pl.BlockSpec((ROW_BLK, D), lambda i: (i, 0)),
        pl.BlockSpec((ROW_BLK, D), lambda i: (i, 0)),
        pl.BlockSpec((ROW_BLK, 1), lambda i: (i, 0)),
        pl.BlockSpec((1, D), lambda i: (0, 0)),
    ],
    out_specs=pl.BlockSpec((ROW_BLK, D), lambda i: (i, 0)),
    out_shape=jax.ShapeDtypeStruct((N, D), jnp.float32),
)


def kernel(x, edge_index, W1, b1, W2, b2):
    src = edge_index[0].astype(jnp.int32)
    dst = edge_index[1].astype(jnp.int32)
    srcp = jnp.concatenate([src, jnp.zeros((E_PAD - E,), jnp.int32)])
    dstp = jnp.concatenate([dst, jnp.full((E_PAD - E,), SINK, jnp.int32)])
    srcp = srcp.reshape(NW, CPT, CHUNK)
    dstp = dstp.reshape(NW, CPT, CHUNK)
    zeros1 = jnp.zeros((DEG_PAD,), jnp.float32)
    zeros2 = jnp.zeros((1000, D), jnp.float32)

    deg0, deg1 = _deg_call()(dstp, zeros1)
    h1 = _mm_call(x, W1)
    dinv, g1 = _scale_call(deg0[:N].reshape(N, 1), deg1[:N].reshape(N, 1), h1)
    a10, a11 = _agg_call()(g1, srcp, dstp, zeros2)
    g2 = _layer2_call(a10, a11, g1, dinv, b1.reshape(1, D), W2)
    a20, a21 = _agg_call()(g2, srcp, dstp, zeros2)
    return _out_call(a20, a21, g2, dinv, b2.reshape(1, D))


# speedup vs baseline: 16.2070x; 1.0707x over previous
import functools

import jax
import jax.numpy as jnp
from jax import lax
from jax.experimental import pallas as pl
from jax.experimental.pallas import tpu as pltpu
from jax.experimental.pallas import tpu_sc as plsc

N = 10000
D = 128
E = 320000
TEMP = 0.2

NC = 2
NS = 16
NW = NC * NS

CHUNK = 128
CPT = 79
E_PAD = NW * CPT * CHUNK
SINK = N
ACC_ROWS = 10048
DEG_PAD = 10240
PAIRS = 39

ROW_BLK = 1000


def _deg_body(dstp3, zeros1, out0, out1, idx_d_all, ones_v, sem, acc):
    c = lax.axis_index("c")
    s = lax.axis_index("s")
    w = c * NS + s

    pltpu.sync_copy(zeros1.at[pl.ds(s * 640, 640)],
                    acc.at[pl.ds(s * 640, 640)])
    pltpu.sync_copy(dstp3.at[w], idx_d_all)
    for i in range(CHUNK // 16):
        ones_v[pl.ds(i * 16, 16)] = jnp.ones((16,), jnp.float32)
    plsc.subcore_barrier()

    def body(k, carry):
        pltpu.async_copy(ones_v, acc.at[idx_d_all.at[k]], sem, add=True)
        return carry

    lax.fori_loop(0, CPT, body, 0)

    def drain(k, carry):
        pltpu.make_async_copy(ones_v, acc.at[idx_d_all.at[0]], sem).wait()
        return carry

    lax.fori_loop(0, CPT, drain, 0)
    plsc.subcore_barrier()

    @pl.when(c == 0)
    def _():
        pltpu.sync_copy(acc.at[pl.ds(s * 640, 640)],
                        out0.at[pl.ds(s * 640, 640)])

    @pl.when(c == 1)
    def _():
        pltpu.sync_copy(acc.at[pl.ds(s * 640, 640)],
                        out1.at[pl.ds(s * 640, 640)])


@functools.cache
def _deg_call():
    mesh = plsc.VectorSubcoreMesh(core_axis_name="c", subcore_axis_name="s")
    return pl.kernel(
        _deg_body,
        out_type=[jax.ShapeDtypeStruct((DEG_PAD,), jnp.float32),
                  jax.ShapeDtypeStruct((DEG_PAD,), jnp.float32)],
        mesh=mesh,
        scratch_types=[
            pltpu.VMEM((CPT, CHUNK), jnp.int32),
            pltpu.VMEM((CHUNK,), jnp.float32),
            pltpu.SemaphoreType.DMA,
            pltpu.VMEM_SHARED((DEG_PAD,), jnp.float32),
        ],
    )


def _agg_body(g, srcp3, dstp3, zeros2, out0, out1,
              is0, is1, is2, id0, id1, id2, r0, r1, r2,
              sis0, sis1, sis2, sid0, sid1, sid2,
              sg0, sg1, sg2, ss0, ss1, ss2, acc):
    c = lax.axis_index("c")
    s = lax.axis_index("s")
    w = c * NS + s
    isb = (is0, is1, is2)
    idb = (id0, id1, id2)
    rows = (r0, r1, r2)
    sis = (sis0, sis1, sis2)
    sid = (sid0, sid1, sid2)
    sg = (sg0, sg1, sg2)
    ss = (ss0, ss1, ss2)

    @pl.when(s < 10)
    def _():
        pltpu.sync_copy(zeros2, acc.at[pl.ds(s * 1000, 1000)])

    def start_gather(b):
        pltpu.async_copy(g.at[isb[b]], rows[b], sg[b])

    def wait_gather(b):
        pltpu.make_async_copy(g.at[isb[b]], rows[b], sg[b]).wait()

    def start_scatter(b):
        pltpu.async_copy(rows[b], acc.at[idb[b]], ss[b], add=True)

    def wait_scatter(b):
        pltpu.make_async_copy(rows[b], acc.at[idb[b]], ss[b]).wait()

    def load_is(k, b):
        pltpu.async_copy(srcp3.at[w, k], isb[b], sis[b])

    def wait_is(b):
        pltpu.make_async_copy(srcp3.at[w, 0], isb[b], sis[b]).wait()

    def load_id(k, b):
        pltpu.async_copy(dstp3.at[w, k], idb[b], sid[b])

    def wait_id(b):
        pltpu.make_async_copy(dstp3.at[w, 0], idb[b], sid[b]).wait()

    pltpu.sync_copy(srcp3.at[w, 0], is0)
    pltpu.sync_copy(srcp3.at[w, 1], is1)
    pltpu.sync_copy(srcp3.at[w, 2], is2)
    pltpu.sync_copy(dstp3.at[w, 0], id0)
    pltpu.async_copy(dstp3.at[w, 1], id1, sid1)
    plsc.subcore_barrier()
    start_gather(0)
    start_gather(1)
    wait_gather(0)
    load_is(3, 0)
    start_scatter(0)
    load_id(2, 2)
    start_gather(2)

    def triple(t, carry):
        for j in range(3):
            k = 3 * t + 1 + j
            b = (1 + j) % 3
            pb = j % 3
            wait_gather(b)

            @pl.when(t < 25)
            def _(k=k, b=b):
                load_is(k + 3, b)

            wait_id(b)
            start_scatter(b)
            wait_scatter(pb)
            if j == 0:
                load_id(k + 2, pb)
                wait_is(pb)
                start_gather(pb)
            else:
                @pl.when(t < 25)
                def _(k=k, pb=pb):
                    load_id(k + 2, pb)
                    wait_is(pb)
                    start_gather(pb)
        return carry

    lax.fori_loop(0, 26, triple, 0)
    wait_scatter(0)
    plsc.subcore_barrier()

    @pl.when(jnp.logical_and(c == 0, s < 10))
    def _():
        pltpu.sync_copy(acc.at[pl.ds(s * 1000, 1000)],
                        out0.at[pl.ds(s * 1000, 1000)])

    @pl.when(jnp.logical_and(c == 1, s < 10))
    def _():
        pltpu.sync_copy(acc.at[pl.ds(s * 1000, 1000)],
                        out1.at[pl.ds(s * 1000, 1000)])


@functools.cache
def _agg_call():
    mesh = plsc.VectorSubcoreMesh(core_axis_name="c", subcore_axis_name="s")
    return pl.kernel(
        _agg_body,
        out_type=[jax.ShapeDtypeStruct((N, D), jnp.float32),
                  jax.ShapeDtypeStruct((N, D), jnp.float32)],
        mesh=mesh,
        scratch_types=(
            [pltpu.VMEM((CHUNK,), jnp.int32)] * 6
            + [pltpu.VMEM((CHUNK, D), jnp.float32)] * 3
            + [pltpu.SemaphoreType.DMA] * 12
            + [pltpu.VMEM_SHARED((ACC_ROWS, D), jnp.float32)]
        ),
    )


def _mm_body(x_ref, w_ref, o_ref):
    o_ref[...] = jnp.dot(x_ref[...], w_ref[...],
                         precision=lax.Precision.HIGHEST,
                         preferred_element_type=jnp.float32)


_mm_call = pl.pallas_call(
    _mm_body,
    grid=(N // ROW_BLK,),
    in_specs=[
        pl.BlockSpec((ROW_BLK, D), lambda i: (i, 0)),
        pl.BlockSpec((D, D), lambda i: (0, 0)),
    ],
    out_specs=pl.BlockSpec((ROW_BLK, D), lambda i: (i, 0)),
    out_shape=jax.ShapeDtypeStruct((N, D), jnp.float32),
)


def _scale_body(d0_ref, d1_ref, h_ref, dinv_ref, g_ref):
    deg = d0_ref[...] + d1_ref[...] + 1.0
    dinv = lax.rsqrt(deg)
    dinv_ref[...] = dinv
    g_ref[...] = h_ref[...] * dinv


_scale_call = pl.pallas_call(
    _scale_body,
    grid=(N // ROW_BLK,),
    in_specs=[
        pl.BlockSpec((ROW_BLK, 1), lambda i: (i, 0)),
        pl.BlockSpec((ROW_BLK, 1), lambda i: (i, 0)),
        pl.BlockSpec((ROW_BLK, D), lambda i: (i, 0)),
    ],
    out_specs=[
        pl.BlockSpec((ROW_BLK, 1), lambda i: (i, 0)),
        pl.BlockSpec((ROW_BLK, D), lambda i: (i, 0)),
    ],
    out_shape=[
        jax.ShapeDtypeStruct((N, 1), jnp.float32),
        jax.ShapeDtypeStruct((N, D), jnp.float32),
    ],
)


def _layer2_body(a0_ref, a1_ref, g1_ref, dinv_ref, b1_ref, w2_ref, g2_ref):
    acc = a0_ref[...] + a1_ref[...]
    z = jnp.maximum(dinv_ref[...] * (acc + g1_ref[...]) + b1_ref[...], 0.0)
    h2 = jnp.dot(z, w2_ref[...],
                 precision=lax.Precision.HIGHEST,
                 preferred_element_type=jnp.float32)
    g2_ref[...] = h2 * dinv_ref[...]


_layer2_call = pl.pallas_call(
    _layer2_body,
    grid=(N // ROW_BLK,),
    in_specs=[
        pl.BlockSpec((ROW_BLK, D), lambda i: (i, 0)),
        pl.BlockSpec((ROW_BLK, D), lambda i: (i, 0)),
        pl.BlockSpec((ROW_BLK, D), lambda i: (i, 0)),
        pl.BlockSpec((ROW_BLK, 1), lambda i: (i, 0)),
        pl.BlockSpec((1, D), lambda i: (0, 0)),
        pl.BlockSpec((D, D), lambda i: (0, 0)),
    ],
    out_specs=pl.BlockSpec((ROW_BLK, D), lambda i: (i, 0)),
    out_shape=jax.ShapeDtypeStruct((N, D), jnp.float32),
)


def _out_body(a0_ref, a1_ref, g2_ref, dinv_ref, b2_ref, o_ref):
    acc = a0_ref[...] + a1_ref[...]
    y = (dinv_ref[...] * (acc + g2_ref[...]) + b2_ref[...]) / TEMP
    m = jnp.max(y, axis=1, keepdims=True)
    lse = jnp.log(jnp.sum(jnp.exp(y - m), axis=1, keepdims=True)) + m
    o_ref[...] = y - lse


_out_call = pl.pallas_call(
    _out_body,
    grid=(N // ROW_BLK,),
    in_specs=[
        pl.BlockSpec((ROW_BLK, D), lambda i: (i, 0)),
        pl.BlockSpec((ROW_BLK, D), lambda i: (i, 0)),
        pl.BlockSpec((ROW_BLK, D), lambda i: (i, 0)),
        pl.BlockSpec((ROW_BLK, 1), lambda i: (i, 0)),
        pl.BlockSpec((1, D), lambda i: (0, 0)),
    ],
    out_specs=pl.BlockSpec((ROW_BLK, D), lambda i: (i, 0)),
    out_shape=jax.ShapeDtypeStruct((N, D), jnp.float32),
)


def kernel(x, edge_index, W1, b1, W2, b2):
    src = edge_index[0].astype(jnp.int32)
    dst = edge_index[1].astype(jnp.int32)
    srcp = jnp.concatenate([src, jnp.zeros((E_PAD - E,), jnp.int32)])
    dstp = jnp.concatenate([dst, jnp.full((E_PAD - E,), SINK, jnp.int32)])
    srcp = srcp.reshape(NW, CPT, CHUNK)
    dstp = dstp.reshape(NW, CPT, CHUNK)
    zeros1 = jnp.zeros((DEG_PAD,), jnp.float32)
    zeros2 = jnp.zeros((1000, D), jnp.float32)

    deg0, deg1 = _deg_call()(dstp, zeros1)
    h1 = _mm_call(x, W1)
    dinv, g1 = _scale_call(deg0[:N].reshape(N, 1), deg1[:N].reshape(N, 1), h1)
    a10, a11 = _agg_call()(g1, srcp, dstp, zeros2)
    g2 = _layer2_call(a10, a11, g1, dinv, b1.reshape(1, D), W2)
    a20, a21 = _agg_call()(g2, srcp, dstp, zeros2)
    return _out_call(a20, a21, g2, dinv, b2.reshape(1, D))
